# baseline (speedup 1.0000x reference)
import functools

import jax
import jax.numpy as jnp
from jax import lax
from jax.experimental import pallas as pl
from jax.experimental.pallas import tpu as pltpu
from jax.experimental.pallas import tpu_sc as plsc

N_NODES = 10000
HID = 8
HPAD = 16
NC = 2
NS = 16
NW = NC * NS
CH = 512
NROWS = 10240
RPS = NROWS // NS
DUMMY = N_NODES


def _sc_degrees(src3d, dst3d):
    ncv = src3d.shape[1]
    mesh = plsc.VectorSubcoreMesh(core_axis_name="c", subcore_axis_name="s")

    @functools.partial(
        pl.kernel,
        mesh=mesh,
        out_type=(
            jax.ShapeDtypeStruct((NC * NROWS,), jnp.float32),
            jax.ShapeDtypeStruct((NC * NROWS,), jnp.float32),
        ),
        scratch_types=[
            pltpu.VMEM((ncv, CH), jnp.int32),
            pltpu.VMEM((ncv, CH), jnp.int32),
            pltpu.VMEM((CH,), jnp.float32),
            pltpu.VMEM((RPS,), jnp.float32),
            pltpu.VMEM_SHARED((NROWS,), jnp.float32),
            pltpu.VMEM_SHARED((NROWS,), jnp.float32),
            pltpu.SemaphoreType.DMA,
        ],
        compiler_params=pltpu.CompilerParams(use_tc_tiling_on_sc=False, skip_device_barrier=True),
    )
    def k(src_h, dst_h, do_h, di_h, srcv, dstv, onesv, zbuf, degA, degB,
          dsem):
        c = lax.axis_index("c")
        s = lax.axis_index("s")
        wid = c * NS + s

        def fill(i, carry):
            onesv[pl.ds(i * 16, 16)] = jnp.ones((16,), jnp.float32)
            return carry

        lax.fori_loop(0, CH // 16, fill, 0)

        def zfill(i, carry):
            zbuf[pl.ds(i * 16, 16)] = jnp.zeros((16,), jnp.float32)
            return carry

        lax.fori_loop(0, RPS // 16, zfill, 0)
        pltpu.sync_copy(zbuf, degA.at[pl.ds(s * RPS, RPS)])
        pltpu.sync_copy(zbuf, degB.at[pl.ds(s * RPS, RPS)])
        pltpu.sync_copy(src_h.at[wid], srcv)
        pltpu.sync_copy(dst_h.at[wid], dstv)
        plsc.subcore_barrier()

        def body(j, carry):
            pltpu.async_copy(onesv, degA.at[srcv.at[j]], dsem, add=True)
            pltpu.async_copy(onesv, degB.at[dstv.at[j]], dsem, add=True)
            return carry

        lax.fori_loop(0, ncv, body, 0)

        def drain(j, carry):
            pltpu.make_async_copy(onesv, degA.at[srcv.at[j]], dsem).wait()
            pltpu.make_async_copy(onesv, degB.at[dstv.at[j]], dsem).wait()
            return carry

        lax.fori_loop(0, ncv, drain, 0)
        plsc.subcore_barrier()
        pltpu.sync_copy(degA.at[pl.ds(s * RPS, RPS)],
                        do_h.at[pl.ds(c * NROWS + s * RPS, RPS)])
        pltpu.sync_copy(degB.at[pl.ds(s * RPS, RPS)],
                        di_h.at[pl.ds(c * NROWS + s * RPS, RPS)])

    return k(src3d, dst3d)


K_GRP = 5
K_GRP2 = 2


def _edge_pipeline(tab, srcv, dstv, rowsv, agg, gsA, gsB, ssA, ssB, ncv, kgrp):
    pairs = ncv // (2 * kgrp)
    assert ncv == pairs * 2 * kgrp

    def pair(p, carry):
        for par, gsem, ssem in ((0, gsA, ssA), (1, gsB, ssB)):
            o = 2 * p + par

            @pl.when(p >= 1)
            def _drain_old():
                for b in range(kgrp):
                    g_old = (o - 2) * kgrp + b
                    pltpu.make_async_copy(rowsv.at[par * kgrp + b],
                                          agg.at[dstv.at[g_old]], ssem).wait()

            for b in range(kgrp):
                g = o * kgrp + b
                pltpu.async_copy(tab.at[srcv.at[g]],
                                 rowsv.at[par * kgrp + b], gsem)
            for b in range(kgrp):
                g = o * kgrp + b
                pltpu.make_async_copy(tab.at[srcv.at[g]],
                                      rowsv.at[par * kgrp + b], gsem).wait()
            for b in range(kgrp):
                g = o * kgrp + b
                pltpu.async_copy(rowsv.at[par * kgrp + b],
                                 agg.at[dstv.at[g]], ssem, add=True)
        return carry

    lax.fori_loop(0, pairs, pair, 0)
    for par, ssem in ((0, ssA), (1, ssB)):
        o = (pairs - 1) * 2 + par
        for b in range(kgrp):
            g = o * kgrp + b
            pltpu.make_async_copy(rowsv.at[par * kgrp + b],
                                  agg.at[dstv.at[g]], ssem).wait()


def _zero_rows(zbuf):
    def zfill(i, carry):
        zbuf[i, :] = jnp.zeros((16,), jnp.float32)
        return carry

    lax.fori_loop(0, RPS, zfill, 0)


def _sc_msgpass(table, src3d, dst3d):
    ncv = src3d.shape[1]
    mesh = plsc.VectorSubcoreMesh(core_axis_name="c", subcore_axis_name="s")

    @functools.partial(
        pl.kernel,
        mesh=mesh,
        out_type=jax.ShapeDtypeStruct((NC * NROWS, HPAD), jnp.float32),
        scratch_types=[
            pltpu.VMEM((ncv, CH), jnp.int32),
            pltpu.VMEM((ncv, CH), jnp.int32),
            pltpu.VMEM((2 * K_GRP, CH, HPAD), jnp.float32),
            pltpu.VMEM((RPS, HPAD), jnp.float32),
            pltpu.VMEM_SHARED((NROWS, HPAD), jnp.float32),
            pltpu.SemaphoreType.DMA,
            pltpu.SemaphoreType.DMA,
            pltpu.SemaphoreType.DMA,
            pltpu.SemaphoreType.DMA,
        ],
        compiler_params=pltpu.CompilerParams(use_tc_tiling_on_sc=False, skip_device_barrier=True),
    )
    def k(tab_h, src_h, dst_h, agg_h, srcv, dstv, rowsv, zbuf, agg,
          gsA, gsB, ssA, ssB):
        c = lax.axis_index("c")
        s = lax.axis_index("s")
        wid = c * NS + s
        _zero_rows(zbuf)
        pltpu.sync_copy(zbuf, agg.at[pl.ds(s * RPS, RPS)])
        pltpu.sync_copy(src_h.at[wid], srcv)
        pltpu.sync_copy(dst_h.at[wid], dstv)
        plsc.subcore_barrier()
        _edge_pipeline(tab_h, srcv, dstv, rowsv, agg, gsA, gsB, ssA, ssB,
                       ncv, K_GRP)
        plsc.subcore_barrier()
        pltpu.sync_copy(agg.at[pl.ds(s * RPS, RPS)],
                        agg_h.at[pl.ds(c * NROWS + s * RPS, RPS)])

    return k(table, src3d, dst3d)


def _sc_msgpass2(p1, h0p, m8, src3d, dst3d):
    ncv = src3d.shape[1]
    mesh = plsc.VectorSubcoreMesh(core_axis_name="c", subcore_axis_name="s")

    @functools.partial(
        pl.kernel,
        mesh=mesh,
        out_type=jax.ShapeDtypeStruct((NC * NROWS, HPAD), jnp.float32),
        scratch_types=[
            pltpu.VMEM((ncv, CH), jnp.int32),
            pltpu.VMEM((ncv, CH), jnp.int32),
            pltpu.VMEM((2 * K_GRP2, CH, HPAD), jnp.float32),
            pltpu.VMEM((RPS, HPAD), jnp.float32),
            pltpu.VMEM((RPS, HPAD), jnp.float32),
            pltpu.VMEM((RPS, HPAD), jnp.float32),
            pltpu.VMEM((RPS, HPAD), jnp.float32),
            pltpu.VMEM_SHARED((NROWS, HPAD), jnp.float32),
            pltpu.VMEM_SHARED((NROWS, HPAD), jnp.float32),
            pltpu.SemaphoreType.DMA,
            pltpu.SemaphoreType.DMA,
            pltpu.SemaphoreType.DMA,
            pltpu.SemaphoreType.DMA,
        ],
        compiler_params=pltpu.CompilerParams(use_tc_tiling_on_sc=False, skip_device_barrier=True),
    )
    def k(p_h, h0_h, m8_h, src_h, dst_h, q_h, srcv, dstv, rowsv,
          cbuf, t1, t2, t3, h1m, agg, gsA, gsB, ssA, ssB):
        c = lax.axis_index("c")
        s = lax.axis_index("s")
        wid = c * NS + s
        base = s * RPS
        _zero_rows(t1)
        pltpu.sync_copy(t1, agg.at[pl.ds(base, RPS)])
        pltpu.sync_copy(src_h.at[wid], srcv)
        pltpu.sync_copy(dst_h.at[wid], dstv)
        pltpu.async_copy(p_h.at[pl.ds(base, RPS)], cbuf, gsA)
        pltpu.async_copy(p_h.at[pl.ds(NROWS + base, RPS)], t1, gsA)
        pltpu.async_copy(h0_h.at[pl.ds(base, RPS)], t2, gsA)
        pltpu.async_copy(m8_h.at[pl.ds(base, RPS)], t3, gsA)
        pltpu.make_async_copy(p_h.at[pl.ds(base, RPS)], cbuf, gsA).wait()
        pltpu.make_async_copy(p_h.at[pl.ds(NROWS + base, RPS)], t1, gsA).wait()
        pltpu.make_async_copy(h0_h.at[pl.ds(base, RPS)], t2, gsA).wait()
        pltpu.make_async_copy(m8_h.at[pl.ds(base, RPS)], t3, gsA).wait()

        def step(i, carry):
            cbuf[i, :] = (cbuf[i, :] + t1[i, :] + t2[i, :]) * t3[i, :]
            return carry

        lax.fori_loop(0, RPS, step, 0)
        pltpu.sync_copy(cbuf, h1m.at[pl.ds(base, RPS)])
        plsc.subcore_barrier()
        _edge_pipeline(h1m, srcv, dstv, rowsv, agg, gsA, gsB, ssA, ssB,
                       ncv, K_GRP2)
        plsc.subcore_barrier()
        pltpu.sync_copy(agg.at[pl.ds(base, RPS)],
                        q_h.at[pl.ds(c * NROWS + base, RPS)])

    return k(p1, h0p, m8, src3d, dst3d)


def _tc_prep(x, W1p, degp_o, degp_i):
    def body(x_r, w_r, do_r, di_r, h_r, m_r, ni_r):
        no = lax.rsqrt(do_r[0, :] + do_r[1, :] + 1.0)
        ni = lax.rsqrt(di_r[0, :] + di_r[1, :] + 1.0)
        m_r[...] = jnp.broadcast_to((no * ni)[:, None], (NROWS, HPAD))
        ni_r[...] = ni[:, None]
        h = jnp.dot(x_r[...] * no[:N_NODES, None], w_r[...],
                    preferred_element_type=jnp.float32)
        h_r[...] = jnp.concatenate(
            [h, jnp.zeros((NROWS - N_NODES, HPAD), jnp.float32)], axis=0)

    return pl.pallas_call(
        body,
        out_shape=(
            jax.ShapeDtypeStruct((NROWS, HPAD), jnp.float32),
            jax.ShapeDtypeStruct((NROWS, HPAD), jnp.float32),
            jax.ShapeDtypeStruct((NROWS, 1), jnp.float32),
        ),
    )(x, W1p, degp_o, degp_i)


def _tc_fin(q, p1, h0p, m8, ni, W2p):
    def body(q_r, p_r, h_r, m_r, ni_r, w_r, o_r):
        h1m = (p_r[0] + p_r[1] + h_r[...]) * m_r[...]
        pre = (q_r[0] + q_r[1] + h1m) * ni_r[...]
        o_r[...] = jnp.dot(pre, w_r[...], preferred_element_type=jnp.float32)

    return pl.pallas_call(
        body,
        out_shape=jax.ShapeDtypeStruct((NROWS, HID), jnp.float32),
    )(q, p1, h0p, m8, ni, W2p)


def kernel(x, edge_index, W1, W2):
    e = edge_index.shape[1]
    ncv = -(-e // (NW * CH))
    lcm = 2 * K_GRP * K_GRP2
    ncv = -(-ncv // lcm) * lcm
    e_pad = NW * ncv * CH
    pad = DUMMY + (jnp.arange(e_pad - e, dtype=jnp.int32) % (NROWS - N_NODES))
    src3d = jnp.concatenate([edge_index[0], pad]).reshape(NW, ncv, CH)
    dst3d = jnp.concatenate([edge_index[1], pad]).reshape(NW, ncv, CH)
    W1p = jnp.pad(W1, ((0, 0), (0, HPAD - HID)))
    W2p = jnp.pad(W2, ((0, HPAD - HID), (0, 0)))

    degp_o, degp_i = _sc_degrees(src3d, dst3d)
    degp_o = degp_o.reshape(NC, NROWS)
    degp_i = degp_i.reshape(NC, NROWS)
    h0p, m8, ni = _tc_prep(x, W1p, degp_o, degp_i)
    p1 = _sc_msgpass(h0p, src3d, dst3d)
    q = _sc_msgpass2(p1, h0p, m8, src3d, dst3d)
    out = _tc_fin(q.reshape(NC, NROWS, HPAD), p1.reshape(NC, NROWS, HPAD),
                  h0p, m8, ni, W2p)
    return out[:N_NODES]


# speedup vs baseline: 1.0011x; 1.0011x over previous
import functools

import jax
import jax.numpy as jnp
from jax import lax
from jax.experimental import pallas as pl
from jax.experimental.pallas import tpu as pltpu
from jax.experimental.pallas import tpu_sc as plsc

N_NODES = 10000
HID = 8
HPAD = 16
NC = 2
NS = 16
NW = NC * NS
CH = 512
NROWS = 10240
RPS = NROWS // NS
DUMMY = N_NODES


def _sc_degrees(src3d, dst3d):
    ncv = src3d.shape[1]
    mesh = plsc.VectorSubcoreMesh(core_axis_name="c", subcore_axis_name="s")

    @functools.partial(
        pl.kernel,
        mesh=mesh,
        out_type=(
            jax.ShapeDtypeStruct((NC * NROWS,), jnp.float32),
            jax.ShapeDtypeStruct((NC * NROWS,), jnp.float32),
        ),
        scratch_types=[
            pltpu.VMEM((ncv, CH), jnp.int32),
            pltpu.VMEM((ncv, CH), jnp.int32),
            pltpu.VMEM((CH,), jnp.float32),
            pltpu.VMEM((RPS,), jnp.float32),
            pltpu.VMEM_SHARED((NROWS,), jnp.float32),
            pltpu.VMEM_SHARED((NROWS,), jnp.float32),
            pltpu.SemaphoreType.DMA,
        ],
        compiler_params=pltpu.CompilerParams(use_tc_tiling_on_sc=False),
    )
    def k(src_h, dst_h, do_h, di_h, srcv, dstv, onesv, zbuf, degA, degB,
          dsem):
        c = lax.axis_index("c")
        s = lax.axis_index("s")
        wid = c * NS + s

        def fill(i, carry):
            onesv[pl.ds(i * 16, 16)] = jnp.ones((16,), jnp.float32)
            return carry

        lax.fori_loop(0, CH // 16, fill, 0)

        def zfill(i, carry):
            zbuf[pl.ds(i * 16, 16)] = jnp.zeros((16,), jnp.float32)
            return carry

        lax.fori_loop(0, RPS // 16, zfill, 0)
        pltpu.sync_copy(zbuf, degA.at[pl.ds(s * RPS, RPS)])
        pltpu.sync_copy(zbuf, degB.at[pl.ds(s * RPS, RPS)])
        pltpu.sync_copy(src_h.at[wid], srcv)
        pltpu.sync_copy(dst_h.at[wid], dstv)
        plsc.subcore_barrier()

        def body(j, carry):
            pltpu.async_copy(onesv, degA.at[srcv.at[j]], dsem, add=True)
            pltpu.async_copy(onesv, degB.at[dstv.at[j]], dsem, add=True)
            return carry

        lax.fori_loop(0, ncv, body, 0)

        def drain(j, carry):
            pltpu.make_async_copy(onesv, degA.at[srcv.at[j]], dsem).wait()
            pltpu.make_async_copy(onesv, degB.at[dstv.at[j]], dsem).wait()
            return carry

        lax.fori_loop(0, ncv, drain, 0)
        plsc.subcore_barrier()
        pltpu.sync_copy(degA.at[pl.ds(s * RPS, RPS)],
                        do_h.at[pl.ds(c * NROWS + s * RPS, RPS)])
        pltpu.sync_copy(degB.at[pl.ds(s * RPS, RPS)],
                        di_h.at[pl.ds(c * NROWS + s * RPS, RPS)])

    return k(src3d, dst3d)


K_GRP = 5
K_GRP2 = 2


def _edge_pipeline(tab, srcv, dstv, rowsv, agg, gsA, gsB, ssA, ssB, ncv, kgrp):
    pairs = ncv // (2 * kgrp)
    assert ncv == pairs * 2 * kgrp

    def pair(p, carry):
        for par, gsem, ssem in ((0, gsA, ssA), (1, gsB, ssB)):
            o = 2 * p + par

            @pl.when(p >= 1)
            def _drain_old():
                for b in range(kgrp):
                    g_old = (o - 2) * kgrp + b
                    pltpu.make_async_copy(rowsv.at[par * kgrp + b],
                                          agg.at[dstv.at[g_old]], ssem).wait()

            for b in range(kgrp):
                g = o * kgrp + b
                pltpu.async_copy(tab.at[srcv.at[g]],
                                 rowsv.at[par * kgrp + b], gsem)
            for b in range(kgrp):
                g = o * kgrp + b
                pltpu.make_async_copy(tab.at[srcv.at[g]],
                                      rowsv.at[par * kgrp + b], gsem).wait()
            for b in range(kgrp):
                g = o * kgrp + b
                pltpu.async_copy(rowsv.at[par * kgrp + b],
                                 agg.at[dstv.at[g]], ssem, add=True)
        return carry

    lax.fori_loop(0, pairs, pair, 0)
    for par, ssem in ((0, ssA), (1, ssB)):
        o = (pairs - 1) * 2 + par
        for b in range(kgrp):
            g = o * kgrp + b
            pltpu.make_async_copy(rowsv.at[par * kgrp + b],
                                  agg.at[dstv.at[g]], ssem).wait()


def _zero_rows(zbuf):
    def zfill(i, carry):
        zbuf[i, :] = jnp.zeros((16,), jnp.float32)
        return carry

    lax.fori_loop(0, RPS, zfill, 0)


def _sc_msgpass(table, src3d, dst3d):
    ncv = src3d.shape[1]
    mesh = plsc.VectorSubcoreMesh(core_axis_name="c", subcore_axis_name="s")

    @functools.partial(
        pl.kernel,
        mesh=mesh,
        out_type=jax.ShapeDtypeStruct((NC * NROWS, HPAD), jnp.float32),
        scratch_types=[
            pltpu.VMEM((ncv, CH), jnp.int32),
            pltpu.VMEM((ncv, CH), jnp.int32),
            pltpu.VMEM((2 * K_GRP, CH, HPAD), jnp.float32),
            pltpu.VMEM((RPS, HPAD), jnp.float32),
            pltpu.VMEM_SHARED((NROWS, HPAD), jnp.float32),
            pltpu.SemaphoreType.DMA,
            pltpu.SemaphoreType.DMA,
            pltpu.SemaphoreType.DMA,
            pltpu.SemaphoreType.DMA,
        ],
        compiler_params=pltpu.CompilerParams(use_tc_tiling_on_sc=False),
    )
    def k(tab_h, src_h, dst_h, agg_h, srcv, dstv, rowsv, zbuf, agg,
          gsA, gsB, ssA, ssB):
        c = lax.axis_index("c")
        s = lax.axis_index("s")
        wid = c * NS + s
        _zero_rows(zbuf)
        pltpu.sync_copy(zbuf, agg.at[pl.ds(s * RPS, RPS)])
        pltpu.sync_copy(src_h.at[wid], srcv)
        pltpu.sync_copy(dst_h.at[wid], dstv)
        plsc.subcore_barrier()
        _edge_pipeline(tab_h, srcv, dstv, rowsv, agg, gsA, gsB, ssA, ssB,
                       ncv, K_GRP)
        plsc.subcore_barrier()
        pltpu.sync_copy(agg.at[pl.ds(s * RPS, RPS)],
                        agg_h.at[pl.ds(c * NROWS + s * RPS, RPS)])

    return k(table, src3d, dst3d)


def _sc_msgpass2(p1, h0p, m8, src3d, dst3d):
    ncv = src3d.shape[1]
    mesh = plsc.VectorSubcoreMesh(core_axis_name="c", subcore_axis_name="s")

    @functools.partial(
        pl.kernel,
        mesh=mesh,
        out_type=jax.ShapeDtypeStruct((NC * NROWS, HPAD), jnp.float32),
        scratch_types=[
            pltpu.VMEM((ncv, CH), jnp.int32),
            pltpu.VMEM((ncv, CH), jnp.int32),
            pltpu.VMEM((2 * K_GRP2, CH, HPAD), jnp.float32),
            pltpu.VMEM((RPS, HPAD), jnp.float32),
            pltpu.VMEM((RPS, HPAD), jnp.float32),
            pltpu.VMEM((RPS, HPAD), jnp.float32),
            pltpu.VMEM((RPS, HPAD), jnp.float32),
            pltpu.VMEM_SHARED((NROWS, HPAD), jnp.float32),
            pltpu.VMEM_SHARED((NROWS, HPAD), jnp.float32),
            pltpu.SemaphoreType.DMA,
            pltpu.SemaphoreType.DMA,
            pltpu.SemaphoreType.DMA,
            pltpu.SemaphoreType.DMA,
        ],
        compiler_params=pltpu.CompilerParams(use_tc_tiling_on_sc=False),
    )
    def k(p_h, h0_h, m8_h, src_h, dst_h, q_h, srcv, dstv, rowsv,
          cbuf, t1, t2, t3, h1m, agg, gsA, gsB, ssA, ssB):
        c = lax.axis_index("c")
        s = lax.axis_index("s")
        wid = c * NS + s
        base = s * RPS
        _zero_rows(t1)
        pltpu.sync_copy(t1, agg.at[pl.ds(base, RPS)])
        pltpu.sync_copy(src_h.at[wid], srcv)
        pltpu.sync_copy(dst_h.at[wid], dstv)
        pltpu.async_copy(p_h.at[pl.ds(base, RPS)], cbuf, gsA)
        pltpu.async_copy(p_h.at[pl.ds(NROWS + base, RPS)], t1, gsA)
        pltpu.async_copy(h0_h.at[pl.ds(base, RPS)], t2, gsA)
        pltpu.async_copy(m8_h.at[pl.ds(base, RPS)], t3, gsA)
        pltpu.make_async_copy(p_h.at[pl.ds(base, RPS)], cbuf, gsA).wait()
        pltpu.make_async_copy(p_h.at[pl.ds(NROWS + base, RPS)], t1, gsA).wait()
        pltpu.make_async_copy(h0_h.at[pl.ds(base, RPS)], t2, gsA).wait()
        pltpu.make_async_copy(m8_h.at[pl.ds(base, RPS)], t3, gsA).wait()

        def step(i, carry):
            cbuf[i, :] = (cbuf[i, :] + t1[i, :] + t2[i, :]) * t3[i, :]
            return carry

        lax.fori_loop(0, RPS, step, 0)
        pltpu.sync_copy(cbuf, h1m.at[pl.ds(base, RPS)])
        plsc.subcore_barrier()
        _edge_pipeline(h1m, srcv, dstv, rowsv, agg, gsA, gsB, ssA, ssB,
                       ncv, K_GRP2)
        plsc.subcore_barrier()
        pltpu.sync_copy(agg.at[pl.ds(base, RPS)],
                        q_h.at[pl.ds(c * NROWS + base, RPS)])

    return k(p1, h0p, m8, src3d, dst3d)


def _tc_prep(x, W1p, degp_o, degp_i):
    def body(x_r, w_r, do_r, di_r, h_r, m_r, ni_r):
        no = lax.rsqrt(do_r[0, :] + do_r[1, :] + 1.0)
        ni = lax.rsqrt(di_r[0, :] + di_r[1, :] + 1.0)
        m_r[...] = jnp.broadcast_to((no * ni)[:, None], (NROWS, HPAD))
        ni_r[...] = ni[:, None]
        h = jnp.dot(x_r[...] * no[:N_NODES, None], w_r[...],
                    preferred_element_type=jnp.float32)
        h_r[...] = jnp.concatenate(
            [h, jnp.zeros((NROWS - N_NODES, HPAD), jnp.float32)], axis=0)

    return pl.pallas_call(
        body,
        out_shape=(
            jax.ShapeDtypeStruct((NROWS, HPAD), jnp.float32),
            jax.ShapeDtypeStruct((NROWS, HPAD), jnp.float32),
            jax.ShapeDtypeStruct((NROWS, 1), jnp.float32),
        ),
    )(x, W1p, degp_o, degp_i)


def _tc_fin(q, p1, h0p, m8, ni, W2p):
    def body(q_r, p_r, h_r, m_r, ni_r, w_r, o_r):
        h1m = (p_r[0] + p_r[1] + h_r[...]) * m_r[...]
        pre = (q_r[0] + q_r[1] + h1m) * ni_r[...]
        o_r[...] = jnp.dot(pre, w_r[...], preferred_element_type=jnp.float32)

    return pl.pallas_call(
        body,
        out_shape=jax.ShapeDtypeStruct((NROWS, HID), jnp.float32),
    )(q, p1, h0p, m8, ni, W2p)


def kernel(x, edge_index, W1, W2):
    e = edge_index.shape[1]
    ncv = -(-e // (NW * CH))
    lcm = 2 * K_GRP * K_GRP2
    ncv = -(-ncv // lcm) * lcm
    e_pad = NW * ncv * CH
    pad = DUMMY + (jnp.arange(e_pad - e, dtype=jnp.int32) % (NROWS - N_NODES))
    src3d = jnp.concatenate([edge_index[0], pad]).reshape(NW, ncv, CH)
    dst3d = jnp.concatenate([edge_index[1], pad]).reshape(NW, ncv, CH)
    W1p = jnp.pad(W1, ((0, 0), (0, HPAD - HID)))
    W2p = jnp.pad(W2, ((0, HPAD - HID), (0, 0)))

    degp_o, degp_i = _sc_degrees(src3d, dst3d)
    degp_o = degp_o.reshape(NC, NROWS)
    degp_i = degp_i.reshape(NC, NROWS)
    h0p, m8, ni = _tc_prep(x, W1p, degp_o, degp_i)
    p1 = _sc_msgpass(h0p, src3d, dst3d)
    q = _sc_msgpass2(p1, h0p, m8, src3d, dst3d)
    out = _tc_fin(q.reshape(NC, NROWS, HPAD), p1.reshape(NC, NROWS, HPAD),
                  h0p, m8, ni, W2p)
    return out[:N_NODES]
